# Initial kernel scaffold; baseline (speedup 1.0000x reference)
import jax
import jax.numpy as jnp
from jax.experimental import pallas as pl


def kernel(x, W):
    raise NotImplementedError("write your pallas kernel here")


# speedup vs baseline: 7.3583x; 7.3583x over previous
import functools

import numpy as np
import jax
import jax.numpy as jnp
from jax import lax
from jax.experimental import pallas as pl
from jax.experimental.pallas import tpu as pltpu
from jax.experimental.pallas import tpu_sc as plsc

_NUM_FIELDS = 26
_FIELD_DIM = 4000
_TOTAL = _NUM_FIELDS * _FIELD_DIM
_EMBED = 16
_BATCH = 4096
_NPAIRS = _NUM_FIELDS * (_NUM_FIELDS - 1) // 2

_info = plsc.get_sparse_core_info()
_NC, _NS, _L = _info.num_cores, _info.num_subcores, _info.num_lanes
_NW = _NC * _NS
_CHUNK = _BATCH // _NW
_NGRP = _CHUNK // _L

_pi = np.array([i for i in range(_NUM_FIELDS) for j in range(i + 1, _NUM_FIELDS)], np.int32)
_pj = np.array([j for i in range(_NUM_FIELDS) for j in range(i + 1, _NUM_FIELDS)], np.int32)
_MPAD = 328
_META_NP = np.zeros((4, _MPAD), np.int32)
_META_NP[0, :_NPAIRS] = _pi
_META_NP[1, :_NPAIRS] = _pj
_META_NP[2, :_NPAIRS] = (_pj - 1) * _TOTAL + _FIELD_DIM * _pi
_META_NP[3, :_NPAIRS] = _pi * _TOTAL + _FIELD_DIM * _pj
_META_NP = _META_NP.reshape(-1)

_mesh = plsc.VectorSubcoreMesh(core_axis_name="c", subcore_axis_name="s")


@functools.partial(
    pl.kernel,
    mesh=_mesh,
    compiler_params=pltpu.CompilerParams(
        needs_layout_passes=False, use_tc_tiling_on_sc=False
    ),
    out_type=jax.ShapeDtypeStruct((_BATCH, _NPAIRS), jnp.float32),
    scratch_types=[
        pltpu.VMEM((_NUM_FIELDS, _CHUNK), jnp.int32),
        pltpu.VMEM((4 * _MPAD,), jnp.int32),
        pltpu.VMEM((2, _CHUNK), jnp.int32),
        pltpu.VMEM((2, _CHUNK), jnp.int32),
        pltpu.VMEM((2, _CHUNK, _EMBED), jnp.float32),
        pltpu.VMEM((2, _CHUNK, _EMBED), jnp.float32),
        pltpu.VMEM((_CHUNK, _NPAIRS), jnp.float32),
        pltpu.SemaphoreType.DMA,
        pltpu.SemaphoreType.DMA,
    ],
)
def _ffm_sc(w2, xt, meta, out, xcols_v, meta_v, idxa_v, idxb_v, bufa_v, bufb_v, out_v, sem0, sem1):
    wid = lax.axis_index("s") * _NC + lax.axis_index("c")
    base = wid * _CHUNK

    for f in range(_NUM_FIELDS):
        pltpu.sync_copy(xt.at[f, pl.ds(base, _CHUNK)], xcols_v.at[f])
    pltpu.sync_copy(meta, meta_v)

    iota16 = lax.iota(jnp.int32, _L)
    samps = [g * _L + iota16 for g in range(_NGRP)]
    dcols = [jnp.full((_L,), d, jnp.int32) for d in range(_EMBED)]
    sems = (sem0, sem1)

    def fire(p, slot):
        pv = jnp.full((_L,), p, jnp.int32)
        fav = plsc.load_gather(meta_v, [pv])
        fbv = plsc.load_gather(meta_v, [pv + _MPAD])
        cav = plsc.load_gather(meta_v, [pv + 2 * _MPAD])
        cbv = plsc.load_gather(meta_v, [pv + 3 * _MPAD])
        for g in range(_NGRP):
            xa = plsc.load_gather(xcols_v, [fav, samps[g]])
            xb = plsc.load_gather(xcols_v, [fbv, samps[g]])
            idxa_v[slot, pl.ds(g * _L, _L)] = xa + cav
            idxb_v[slot, pl.ds(g * _L, _L)] = xb + cbv
        cpa = pltpu.async_copy(w2.at[idxa_v.at[slot]], bufa_v.at[slot], sems[slot])
        cpb = pltpu.async_copy(w2.at[idxb_v.at[slot]], bufb_v.at[slot], sems[slot])
        return cpa, cpb

    def compute(p, slot):
        pcol = jnp.full((_L,), p, jnp.int32)
        for g in range(_NGRP):
            acc = None
            for d in range(_EMBED):
                a = plsc.load_gather(bufa_v.at[slot], [samps[g], dcols[d]])
                b = plsc.load_gather(bufb_v.at[slot], [samps[g], dcols[d]])
                ab = a * b
                acc = ab if acc is None else acc + ab
            plsc.store_scatter(out_v, [samps[g], pcol], acc)

    fire(0, 0)

    def pair_body(p, _):
        slot = lax.rem(p, 2)
        nxt = lax.rem(p + 1, 2)

        @pl.when(p + 1 < _NPAIRS)
        def _():
            @pl.when(nxt == 0)
            def _():
                fire(p + 1, 0)

            @pl.when(nxt == 1)
            def _():
                fire(p + 1, 1)

        @pl.when(slot == 0)
        def _():
            cpa = pltpu.make_async_copy(w2.at[idxa_v.at[0]], bufa_v.at[0], sem0)
            cpb = pltpu.make_async_copy(w2.at[idxb_v.at[0]], bufb_v.at[0], sem0)
            cpa.wait()
            cpb.wait()
            compute(p, 0)

        @pl.when(slot == 1)
        def _():
            cpa = pltpu.make_async_copy(w2.at[idxa_v.at[1]], bufa_v.at[1], sem1)
            cpb = pltpu.make_async_copy(w2.at[idxb_v.at[1]], bufb_v.at[1], sem1)
            cpa.wait()
            cpb.wait()
            compute(p, 1)

        return _

    lax.fori_loop(0, _NPAIRS, pair_body, None)
    pltpu.sync_copy(out_v, out.at[pl.ds(base, _CHUNK)])


def kernel(x, W):
    xt = x.T
    w2 = W.reshape(_NUM_FIELDS * _TOTAL, _EMBED)
    meta = jnp.asarray(_META_NP)
    return _ffm_sc(w2, xt, meta)
